# baseline (speedup 1.0000x reference)
import functools

import jax
import jax.numpy as jnp
from jax import lax
from jax.experimental import pallas as pl
from jax.experimental.pallas import tpu as pltpu
from jax.experimental.pallas import tpu_sc as plsc

N = 10000
E = 320000
D = 128
NC = 2
NS = 16
NW = NC * NS
EPT = E // NW
K = 128
EPTP = 10112
NCH = EPTP // K
NP = 10240
RPS = NP // NS

_mesh = plsc.VectorSubcoreMesh(core_axis_name="c", subcore_axis_name="s")


def _deg_body(dst_hbm, ones_hbm, zeros_hbm, out_hbm, dst_v, ones_v, deg_sp):
  c = lax.axis_index("c")
  s = lax.axis_index("s")
  wid = c * NS + s
  @pl.when(s == 0)
  def _():
    pltpu.sync_copy(zeros_hbm, deg_sp)
  pltpu.sync_copy(ones_hbm, ones_v)
  pltpu.sync_copy(dst_hbm.at[wid], dst_v)
  plsc.subcore_barrier()

  def chunk(j, carry):
    pltpu.sync_copy(ones_v, deg_sp.at[dst_v.at[j]], add=True)
    return carry
  lax.fori_loop(0, NCH, chunk, 0)

  plsc.subcore_barrier()
  @pl.when(s == 0)
  def _():
    pltpu.sync_copy(deg_sp, out_hbm.at[c])


_deg_call = functools.partial(
    pl.kernel,
    out_type=jax.ShapeDtypeStruct((NC, NP), jnp.float32),
    mesh=_mesh,
    scratch_types=[
        pltpu.VMEM((NCH, K), jnp.int32),
        pltpu.VMEM((K,), jnp.float32),
        pltpu.VMEM_SHARED((NP,), jnp.float32),
    ],
)(_deg_body)


NBUF = 5
ROUNDS = NCH // NBUF


def _prop_body(g_hbm, src_hbm, dst_hbm, zeros_hbm, out_hbm,
               src_v, dst_v, rows_v, acc_sp, gsem, ssem):
  c = lax.axis_index("c")
  s = lax.axis_index("s")
  wid = c * NS + s
  pltpu.sync_copy(zeros_hbm.at[pl.ds(s * RPS, RPS)],
                  acc_sp.at[pl.ds(s * RPS, RPS)])
  pltpu.sync_copy(src_hbm.at[wid], src_v)
  pltpu.sync_copy(dst_hbm.at[wid], dst_v)
  plsc.subcore_barrier()

  def chunk(j, carry):
    pltpu.async_copy(g_hbm.at[src_v.at[j]], rows_v, gsem).wait()
    pltpu.sync_copy(rows_v, acc_sp.at[dst_v.at[j]], add=True)
    return carry
  lax.fori_loop(0, NCH, chunk, 0)

  plsc.subcore_barrier()
  pltpu.sync_copy(acc_sp.at[pl.ds(s * RPS, RPS)],
                  out_hbm.at[c, pl.ds(s * RPS, RPS)])


_prop_call = functools.partial(
    pl.kernel,
    out_type=jax.ShapeDtypeStruct((NC, NP, D), jnp.float32),
    mesh=_mesh,
    scratch_types=[
        pltpu.VMEM((NCH, K), jnp.int32),
        pltpu.VMEM((NCH, K), jnp.int32),
        pltpu.VMEM((K, D), jnp.float32),
        pltpu.VMEM_SHARED((NP, D), jnp.float32),
        pltpu.SemaphoreType.DMA,
        pltpu.SemaphoreType.DMA,
    ],
)(_prop_body)


R = 1000


def _tc_a_body(x_ref, w_ref, degp_ref, g1_ref, dinv_ref):
  d = degp_ref[...]
  deg = 1.0 + d[0] + d[1]
  dinv = lax.rsqrt(deg)
  h = jnp.dot(x_ref[...], w_ref[...], preferred_element_type=jnp.float32)
  g1_ref[...] = h * dinv
  dinv_ref[...] = dinv


def _tc_a(x, W1, degp):
  return pl.pallas_call(
      _tc_a_body,
      grid=(N // R,),
      in_specs=[
          pl.BlockSpec((R, D), lambda i: (i, 0)),
          pl.BlockSpec((D, D), lambda i: (0, 0)),
          pl.BlockSpec((NC, R, 1), lambda i: (0, i, 0)),
      ],
      out_specs=[
          pl.BlockSpec((R, D), lambda i: (i, 0)),
          pl.BlockSpec((R, 1), lambda i: (i, 0)),
      ],
      out_shape=[
          jax.ShapeDtypeStruct((N, D), jnp.float32),
          jax.ShapeDtypeStruct((N, 1), jnp.float32),
      ],
  )(x, W1, degp)


def _tc_b_body(accp_ref, g1_ref, dinv_ref, b1_ref, w2_ref, h1_ref, g2_ref):
  p = accp_ref[...]
  dinv = dinv_ref[...]
  out1 = (p[0] + p[1] + g1_ref[...]) * dinv + b1_ref[...]
  h1 = jnp.maximum(out1, 0.0)
  h1_ref[...] = h1
  h2 = jnp.dot(h1, w2_ref[...], preferred_element_type=jnp.float32)
  g2_ref[...] = h2 * dinv


def _tc_b(accp, g1, dinv, b1, W2):
  return pl.pallas_call(
      _tc_b_body,
      grid=(N // R,),
      in_specs=[
          pl.BlockSpec((NC, R, D), lambda i: (0, i, 0)),
          pl.BlockSpec((R, D), lambda i: (i, 0)),
          pl.BlockSpec((R, 1), lambda i: (i, 0)),
          pl.BlockSpec((1, D), lambda i: (0, 0)),
          pl.BlockSpec((D, D), lambda i: (0, 0)),
      ],
      out_specs=[
          pl.BlockSpec((R, D), lambda i: (i, 0)),
          pl.BlockSpec((R, D), lambda i: (i, 0)),
      ],
      out_shape=[
          jax.ShapeDtypeStruct((N, D), jnp.float32),
          jax.ShapeDtypeStruct((N, D), jnp.float32),
      ],
  )(accp, g1, dinv, b1, W2)


def _tc_c_body(accp_ref, g2_ref, dinv_ref, b2_ref, out_ref):
  p = accp_ref[...]
  dinv = dinv_ref[...]
  out_ref[...] = (p[0] + p[1] + g2_ref[...]) * dinv + b2_ref[...]


def _tc_c(accp, g2, dinv, b2):
  return pl.pallas_call(
      _tc_c_body,
      grid=(N // R,),
      in_specs=[
          pl.BlockSpec((NC, R, D), lambda i: (0, i, 0)),
          pl.BlockSpec((R, D), lambda i: (i, 0)),
          pl.BlockSpec((R, 1), lambda i: (i, 0)),
          pl.BlockSpec((1, D), lambda i: (0, 0)),
      ],
      out_specs=pl.BlockSpec((R, D), lambda i: (i, 0)),
      out_shape=jax.ShapeDtypeStruct((N, D), jnp.float32),
  )(accp, g2, dinv, b2)


@jax.jit
def kernel(x, adj_t, W1, b1, W2, b2):
  npad = EPTP - EPT
  pad_dst = N + (jnp.arange(npad, dtype=jnp.int32) % (NP - N))
  src = adj_t[0].astype(jnp.int32).reshape(NW, EPT)
  dst = adj_t[1].astype(jnp.int32).reshape(NW, EPT)
  src = jnp.pad(src, ((0, 0), (0, npad))).reshape(NW, NCH, K)
  dst = jnp.concatenate(
      [dst, jnp.broadcast_to(pad_dst, (NW, npad))], axis=1
  ).reshape(NW, NCH, K)
  zeros_nd = jnp.zeros((NP, D), jnp.float32)
  zeros_n = jnp.zeros((NP,), jnp.float32)
  ones_k = jnp.ones((K,), jnp.float32)

  degp = _deg_call(dst, ones_k, zeros_n)
  g1, dinv = _tc_a(x, W1, degp[:, :N].reshape(NC, N, 1))
  acc1 = _prop_call(g1, src, dst, zeros_nd)
  h1, g2 = _tc_b(acc1, g1, dinv, b1.reshape(1, D), W2)
  acc2 = _prop_call(g2, src, dst, zeros_nd)
  logit = _tc_c(acc2, g2, dinv, b2.reshape(1, D))
  return (logit, h1)


# speedup vs baseline: 1.5054x; 1.5054x over previous
import functools

import jax
import jax.numpy as jnp
from jax import lax
from jax.experimental import pallas as pl
from jax.experimental.pallas import tpu as pltpu
from jax.experimental.pallas import tpu_sc as plsc

N = 10000
E = 320000
D = 128
DH = D // 2
NC = 2
NS = 16
NW = NC * NS
EPT = E // NW
K = 80
NCH = EPT // K
NP = 10112
RPS = NP // NS
NBUF = 4
ROUNDS = -(-NCH // NBUF)

_mesh = plsc.VectorSubcoreMesh(core_axis_name="c", subcore_axis_name="s")


def _deg_body(dst_hbm, ones_hbm, zeros_hbm, out_hbm, dst_v, ones_v, deg_sp):
  c = lax.axis_index("c")
  s = lax.axis_index("s")
  wid = c * NS + s
  @pl.when(s == 0)
  def _():
    pltpu.sync_copy(zeros_hbm, deg_sp)
  pltpu.sync_copy(ones_hbm, ones_v)
  pltpu.sync_copy(dst_hbm.at[wid], dst_v)
  plsc.subcore_barrier()

  def chunk(j, carry):
    pltpu.sync_copy(ones_v, deg_sp.at[dst_v.at[j]], add=True)
    return carry
  lax.fori_loop(0, NCH, chunk, 0)

  plsc.subcore_barrier()
  @pl.when(s == 0)
  def _():
    pltpu.sync_copy(deg_sp, out_hbm.at[c])


_deg_call = functools.partial(
    pl.kernel,
    out_type=jax.ShapeDtypeStruct((NC, N), jnp.float32),
    mesh=_mesh,
    scratch_types=[
        pltpu.VMEM((NCH, K), jnp.int32),
        pltpu.VMEM((K,), jnp.float32),
        pltpu.VMEM_SHARED((N,), jnp.float32),
    ],
)(_deg_body)


def _prop_body(g_hbm, src_hbm, dst_hbm, zeros_hbm, out_hbm,
               src_v, dst_v, *rest):
  rows = rest[:NBUF]
  acc_sp = rest[NBUF]
  gsems = rest[NBUF + 1:2 * NBUF + 1]
  ssems = rest[2 * NBUF + 1:]
  c = lax.axis_index("c")
  s = lax.axis_index("s")
  wid = c * NS + s
  pltpu.sync_copy(zeros_hbm.at[pl.ds(s * RPS, RPS)],
                  acc_sp.at[pl.ds(s * RPS, RPS)])
  pltpu.sync_copy(src_hbm.at[wid], src_v)
  pltpu.sync_copy(dst_hbm.at[wid], dst_v)
  plsc.subcore_barrier()

  def gather_start(chunk, b):
    pltpu.async_copy(g_hbm.at[src_v.at[chunk]], rows[b], gsems[b])

  def gather_wait(b):
    pltpu.make_async_copy(g_hbm.at[src_v.at[0]], rows[b], gsems[b]).wait()

  def scatter_start(chunk, b):
    pltpu.async_copy(rows[b], acc_sp.at[dst_v.at[chunk]], ssems[b], add=True)

  def scatter_wait(b):
    pltpu.make_async_copy(rows[b], acc_sp.at[dst_v.at[0]], ssems[b]).wait()

  def round_body(rd, carry):
    base = rd * NBUF
    for b in range(NBUF):
      @pl.when(base + b < NCH)
      def _(b=b):
        gather_start(base + b, b)
    for b in range(NBUF):
      @pl.when(base + b < NCH)
      def _(b=b):
        gather_wait(b)
        scatter_start(base + b, b)
    for b in range(NBUF):
      @pl.when(base + b < NCH)
      def _(b=b):
        scatter_wait(b)
    return carry
  lax.fori_loop(0, ROUNDS, round_body, 0)

  plsc.subcore_barrier()
  pltpu.sync_copy(acc_sp.at[pl.ds(s * RPS, RPS)],
                  out_hbm.at[c, pl.ds(s * RPS, RPS)])


_prop_call = functools.partial(
    pl.kernel,
    out_type=jax.ShapeDtypeStruct((NC, NP, DH), jnp.float32),
    mesh=_mesh,
    compiler_params=pltpu.CompilerParams(use_tc_tiling_on_sc=False),
    scratch_types=(
        [pltpu.VMEM((NCH, K), jnp.int32),
         pltpu.VMEM((NCH, K), jnp.int32)]
        + [pltpu.VMEM((K, DH), jnp.float32) for _ in range(NBUF)]
        + [pltpu.VMEM_SHARED((NP, DH), jnp.float32)]
        + [pltpu.SemaphoreType.DMA for _ in range(2 * NBUF)]
    ),
)(_prop_body)


R = 1000


def _tc_a_body(x_ref, w_ref, degp_ref, ga_ref, gb_ref, dinv_ref):
  d = degp_ref[...]
  deg = 1.0 + d[0] + d[1]
  dinv = lax.rsqrt(deg)
  h = jnp.dot(x_ref[...], w_ref[...], preferred_element_type=jnp.float32)
  g = h * dinv
  ga_ref[...] = g[:, :DH]
  gb_ref[...] = g[:, DH:]
  dinv_ref[...] = dinv


def _tc_a(x, W1, degp):
  return pl.pallas_call(
      _tc_a_body,
      grid=(N // R,),
      in_specs=[
          pl.BlockSpec((R, D), lambda i: (i, 0)),
          pl.BlockSpec((D, D), lambda i: (0, 0)),
          pl.BlockSpec((NC, R, 1), lambda i: (0, i, 0)),
      ],
      out_specs=[
          pl.BlockSpec((R, DH), lambda i: (i, 0)),
          pl.BlockSpec((R, DH), lambda i: (i, 0)),
          pl.BlockSpec((R, 1), lambda i: (i, 0)),
      ],
      out_shape=[
          jax.ShapeDtypeStruct((N, DH), jnp.float32),
          jax.ShapeDtypeStruct((N, DH), jnp.float32),
          jax.ShapeDtypeStruct((N, 1), jnp.float32),
      ],
  )(x, W1, degp)


def _tc_b_body(pa_ref, pb_ref, ga_ref, gb_ref, dinv_ref, b1_ref, w2_ref,
               h1_ref, ga2_ref, gb2_ref):
  pa = pa_ref[...]
  pb = pb_ref[...]
  dinv = dinv_ref[...]
  sa = pa[0] + pa[1] + ga_ref[...]
  sb = pb[0] + pb[1] + gb_ref[...]
  out1 = jnp.concatenate([sa, sb], axis=1) * dinv + b1_ref[...]
  h1 = jnp.maximum(out1, 0.0)
  h1_ref[...] = h1
  h2 = jnp.dot(h1, w2_ref[...], preferred_element_type=jnp.float32)
  g2 = h2 * dinv
  ga2_ref[...] = g2[:, :DH]
  gb2_ref[...] = g2[:, DH:]


def _tc_b(pa, pb, ga, gb, dinv, b1, W2):
  return pl.pallas_call(
      _tc_b_body,
      grid=(N // R,),
      in_specs=[
          pl.BlockSpec((NC, R, DH), lambda i: (0, i, 0)),
          pl.BlockSpec((NC, R, DH), lambda i: (0, i, 0)),
          pl.BlockSpec((R, DH), lambda i: (i, 0)),
          pl.BlockSpec((R, DH), lambda i: (i, 0)),
          pl.BlockSpec((R, 1), lambda i: (i, 0)),
          pl.BlockSpec((1, D), lambda i: (0, 0)),
          pl.BlockSpec((D, D), lambda i: (0, 0)),
      ],
      out_specs=[
          pl.BlockSpec((R, D), lambda i: (i, 0)),
          pl.BlockSpec((R, DH), lambda i: (i, 0)),
          pl.BlockSpec((R, DH), lambda i: (i, 0)),
      ],
      out_shape=[
          jax.ShapeDtypeStruct((N, D), jnp.float32),
          jax.ShapeDtypeStruct((N, DH), jnp.float32),
          jax.ShapeDtypeStruct((N, DH), jnp.float32),
      ],
  )(pa, pb, ga, gb, dinv, b1, W2)


def _tc_c_body(pa_ref, pb_ref, ga_ref, gb_ref, dinv_ref, b2_ref, out_ref):
  pa = pa_ref[...]
  pb = pb_ref[...]
  dinv = dinv_ref[...]
  sa = pa[0] + pa[1] + ga_ref[...]
  sb = pb[0] + pb[1] + gb_ref[...]
  out_ref[...] = jnp.concatenate([sa, sb], axis=1) * dinv + b2_ref[...]


def _tc_c(pa, pb, ga, gb, dinv, b2):
  return pl.pallas_call(
      _tc_c_body,
      grid=(N // R,),
      in_specs=[
          pl.BlockSpec((NC, R, DH), lambda i: (0, i, 0)),
          pl.BlockSpec((NC, R, DH), lambda i: (0, i, 0)),
          pl.BlockSpec((R, DH), lambda i: (i, 0)),
          pl.BlockSpec((R, DH), lambda i: (i, 0)),
          pl.BlockSpec((R, 1), lambda i: (i, 0)),
          pl.BlockSpec((1, D), lambda i: (0, 0)),
      ],
      out_specs=pl.BlockSpec((R, D), lambda i: (i, 0)),
      out_shape=jax.ShapeDtypeStruct((N, D), jnp.float32),
  )(pa, pb, ga, gb, dinv, b2)


@jax.jit
def kernel(x, adj_t, W1, b1, W2, b2):
  src = adj_t[0].astype(jnp.int32).reshape(NW, NCH, K)
  dst = adj_t[1].astype(jnp.int32).reshape(NW, NCH, K)
  zeros_hd = jnp.zeros((NP, DH), jnp.float32)
  zeros_n = jnp.zeros((N,), jnp.float32)
  ones_k = jnp.ones((K,), jnp.float32)

  degp = _deg_call(dst, ones_k, zeros_n)
  ga1, gb1, dinv = _tc_a(x, W1, degp.reshape(NC, N, 1))
  pa1 = _prop_call(ga1, src, dst, zeros_hd)
  pb1 = _prop_call(gb1, src, dst, zeros_hd)
  h1, ga2, gb2 = _tc_b(pa1, pb1, ga1, gb1, dinv, b1.reshape(1, D), W2)
  pa2 = _prop_call(ga2, src, dst, zeros_hd)
  pb2 = _prop_call(gb2, src, dst, zeros_hd)
  logit = _tc_c(pa2, pb2, ga2, gb2, dinv, b2.reshape(1, D))
  return (logit, h1)


# speedup vs baseline: 1.6010x; 1.0635x over previous
import functools

import jax
import jax.numpy as jnp
from jax import lax
from jax.experimental import pallas as pl
from jax.experimental.pallas import tpu as pltpu
from jax.experimental.pallas import tpu_sc as plsc

N = 10000
E = 320000
D = 128
DH = D // 2
NC = 2
NS = 16
NW = NC * NS
EPT = E // NW
K = 80
NCH = EPT // K
NP = 10112
RPS = NP // NS
NBUF = 6
ROUNDS = -(-NCH // NBUF)

_mesh = plsc.VectorSubcoreMesh(core_axis_name="c", subcore_axis_name="s")


def _deg_body(dst_hbm, ones_hbm, zeros_hbm, out_hbm, dst_v, ones_v, deg_sp):
  c = lax.axis_index("c")
  s = lax.axis_index("s")
  wid = c * NS + s
  @pl.when(s == 0)
  def _():
    pltpu.sync_copy(zeros_hbm, deg_sp)
  pltpu.sync_copy(ones_hbm, ones_v)
  pltpu.sync_copy(dst_hbm.at[wid], dst_v)
  plsc.subcore_barrier()

  def chunk(j, carry):
    pltpu.sync_copy(ones_v, deg_sp.at[dst_v.at[j]], add=True)
    return carry
  lax.fori_loop(0, NCH, chunk, 0)

  plsc.subcore_barrier()
  @pl.when(s == 0)
  def _():
    pltpu.sync_copy(deg_sp, out_hbm.at[c])


_deg_call = functools.partial(
    pl.kernel,
    out_type=jax.ShapeDtypeStruct((NC, N), jnp.float32),
    mesh=_mesh,
    scratch_types=[
        pltpu.VMEM((NCH, K), jnp.int32),
        pltpu.VMEM((K,), jnp.float32),
        pltpu.VMEM_SHARED((N,), jnp.float32),
    ],
)(_deg_body)


def _prop_body(g_hbm, src_hbm, dst_hbm, zeros_hbm, out_hbm,
               src_v, dst_v, *rest):
  rows = rest[:NBUF]
  acc_sp = rest[NBUF]
  gsems = rest[NBUF + 1:2 * NBUF + 1]
  ssems = rest[2 * NBUF + 1:]
  c = lax.axis_index("c")
  s = lax.axis_index("s")
  wid = c * NS + s
  pltpu.sync_copy(zeros_hbm.at[pl.ds(s * RPS, RPS)],
                  acc_sp.at[pl.ds(s * RPS, RPS)])
  pltpu.sync_copy(src_hbm.at[wid], src_v)
  pltpu.sync_copy(dst_hbm.at[wid], dst_v)
  plsc.subcore_barrier()

  def gather_start(chunk, b):
    pltpu.async_copy(g_hbm.at[src_v.at[chunk]], rows[b], gsems[b])

  def gather_wait(b):
    pltpu.make_async_copy(g_hbm.at[src_v.at[0]], rows[b], gsems[b]).wait()

  def scatter_start(chunk, b):
    pltpu.async_copy(rows[b], acc_sp.at[dst_v.at[chunk]], ssems[b], add=True)

  def scatter_wait(b):
    pltpu.make_async_copy(rows[b], acc_sp.at[dst_v.at[0]], ssems[b]).wait()

  def round_body(rd, carry):
    base = rd * NBUF
    for b in range(NBUF):
      @pl.when(base + b < NCH)
      def _(b=b):
        gather_start(base + b, b)
    for b in range(NBUF):
      @pl.when(base + b < NCH)
      def _(b=b):
        gather_wait(b)
        scatter_start(base + b, b)
    for b in range(NBUF):
      @pl.when(base + b < NCH)
      def _(b=b):
        scatter_wait(b)
    return carry
  lax.fori_loop(0, ROUNDS, round_body, 0)

  plsc.subcore_barrier()
  pltpu.sync_copy(acc_sp.at[pl.ds(s * RPS, RPS)],
                  out_hbm.at[c, pl.ds(s * RPS, RPS)])


_prop_call = functools.partial(
    pl.kernel,
    out_type=jax.ShapeDtypeStruct((NC, NP, DH), jnp.float32),
    mesh=_mesh,
    compiler_params=pltpu.CompilerParams(use_tc_tiling_on_sc=False),
    scratch_types=(
        [pltpu.VMEM((NCH, K), jnp.int32),
         pltpu.VMEM((NCH, K), jnp.int32)]
        + [pltpu.VMEM((K, DH), jnp.float32) for _ in range(NBUF)]
        + [pltpu.VMEM_SHARED((NP, DH), jnp.float32)]
        + [pltpu.SemaphoreType.DMA for _ in range(2 * NBUF)]
    ),
)(_prop_body)


R = 1000


def _tc_a_body(x_ref, w_ref, degp_ref, ga_ref, gb_ref, dinv_ref):
  d = degp_ref[...]
  deg = 1.0 + d[0] + d[1]
  dinv = lax.rsqrt(deg)
  h = jnp.dot(x_ref[...], w_ref[...], preferred_element_type=jnp.float32)
  g = h * dinv
  ga_ref[...] = g[:, :DH]
  gb_ref[...] = g[:, DH:]
  dinv_ref[...] = dinv


def _tc_a(x, W1, degp):
  return pl.pallas_call(
      _tc_a_body,
      grid=(N // R,),
      in_specs=[
          pl.BlockSpec((R, D), lambda i: (i, 0)),
          pl.BlockSpec((D, D), lambda i: (0, 0)),
          pl.BlockSpec((NC, R, 1), lambda i: (0, i, 0)),
      ],
      out_specs=[
          pl.BlockSpec((R, DH), lambda i: (i, 0)),
          pl.BlockSpec((R, DH), lambda i: (i, 0)),
          pl.BlockSpec((R, 1), lambda i: (i, 0)),
      ],
      out_shape=[
          jax.ShapeDtypeStruct((N, DH), jnp.float32),
          jax.ShapeDtypeStruct((N, DH), jnp.float32),
          jax.ShapeDtypeStruct((N, 1), jnp.float32),
      ],
  )(x, W1, degp)


def _tc_b_body(pa_ref, pb_ref, ga_ref, gb_ref, dinv_ref, b1_ref, w2_ref,
               h1_ref, ga2_ref, gb2_ref):
  pa = pa_ref[...]
  pb = pb_ref[...]
  dinv = dinv_ref[...]
  sa = pa[0] + pa[1] + ga_ref[...]
  sb = pb[0] + pb[1] + gb_ref[...]
  out1 = jnp.concatenate([sa, sb], axis=1) * dinv + b1_ref[...]
  h1 = jnp.maximum(out1, 0.0)
  h1_ref[...] = h1
  h2 = jnp.dot(h1, w2_ref[...], preferred_element_type=jnp.float32)
  g2 = h2 * dinv
  ga2_ref[...] = g2[:, :DH]
  gb2_ref[...] = g2[:, DH:]


def _tc_b(pa, pb, ga, gb, dinv, b1, W2):
  return pl.pallas_call(
      _tc_b_body,
      grid=(N // R,),
      in_specs=[
          pl.BlockSpec((NC, R, DH), lambda i: (0, i, 0)),
          pl.BlockSpec((NC, R, DH), lambda i: (0, i, 0)),
          pl.BlockSpec((R, DH), lambda i: (i, 0)),
          pl.BlockSpec((R, DH), lambda i: (i, 0)),
          pl.BlockSpec((R, 1), lambda i: (i, 0)),
          pl.BlockSpec((1, D), lambda i: (0, 0)),
          pl.BlockSpec((D, D), lambda i: (0, 0)),
      ],
      out_specs=[
          pl.BlockSpec((R, D), lambda i: (i, 0)),
          pl.BlockSpec((R, DH), lambda i: (i, 0)),
          pl.BlockSpec((R, DH), lambda i: (i, 0)),
      ],
      out_shape=[
          jax.ShapeDtypeStruct((N, D), jnp.float32),
          jax.ShapeDtypeStruct((N, DH), jnp.float32),
          jax.ShapeDtypeStruct((N, DH), jnp.float32),
      ],
  )(pa, pb, ga, gb, dinv, b1, W2)


def _tc_c_body(pa_ref, pb_ref, ga_ref, gb_ref, dinv_ref, b2_ref, out_ref):
  pa = pa_ref[...]
  pb = pb_ref[...]
  dinv = dinv_ref[...]
  sa = pa[0] + pa[1] + ga_ref[...]
  sb = pb[0] + pb[1] + gb_ref[...]
  out_ref[...] = jnp.concatenate([sa, sb], axis=1) * dinv + b2_ref[...]


def _tc_c(pa, pb, ga, gb, dinv, b2):
  return pl.pallas_call(
      _tc_c_body,
      grid=(N // R,),
      in_specs=[
          pl.BlockSpec((NC, R, DH), lambda i: (0, i, 0)),
          pl.BlockSpec((NC, R, DH), lambda i: (0, i, 0)),
          pl.BlockSpec((R, DH), lambda i: (i, 0)),
          pl.BlockSpec((R, DH), lambda i: (i, 0)),
          pl.BlockSpec((R, 1), lambda i: (i, 0)),
          pl.BlockSpec((1, D), lambda i: (0, 0)),
      ],
      out_specs=pl.BlockSpec((R, D), lambda i: (i, 0)),
      out_shape=jax.ShapeDtypeStruct((N, D), jnp.float32),
  )(pa, pb, ga, gb, dinv, b2)


@jax.jit
def kernel(x, adj_t, W1, b1, W2, b2):
  src = adj_t[0].astype(jnp.int32).reshape(NW, NCH, K)
  dst = adj_t[1].astype(jnp.int32).reshape(NW, NCH, K)
  zeros_hd = jnp.zeros((NP, DH), jnp.float32)
  zeros_n = jnp.zeros((N,), jnp.float32)
  ones_k = jnp.ones((K,), jnp.float32)

  degp = _deg_call(dst, ones_k, zeros_n)
  ga1, gb1, dinv = _tc_a(x, W1, degp.reshape(NC, N, 1))
  pa1 = _prop_call(ga1, src, dst, zeros_hd)
  pb1 = _prop_call(gb1, src, dst, zeros_hd)
  h1, ga2, gb2 = _tc_b(pa1, pb1, ga1, gb1, dinv, b1.reshape(1, D), W2)
  pa2 = _prop_call(ga2, src, dst, zeros_hd)
  pb2 = _prop_call(gb2, src, dst, zeros_hd)
  logit = _tc_c(pa2, pb2, ga2, gb2, dinv, b2.reshape(1, D))
  return (logit, h1)


# speedup vs baseline: 1.6544x; 1.0333x over previous
import functools

import jax
import jax.numpy as jnp
from jax import lax
from jax.experimental import pallas as pl
from jax.experimental.pallas import tpu as pltpu
from jax.experimental.pallas import tpu_sc as plsc

N = 10000
E = 320000
D = 128
DH = D // 2
NC = 2
NS = 16
NW = NC * NS
EPT = E // NW
K = 80
NCH = EPT // K
NP = 10112
RPS = NP // NS
NBUF = 8
ROUNDS = -(-NCH // NBUF)

_mesh = plsc.VectorSubcoreMesh(core_axis_name="c", subcore_axis_name="s")


def _deg_body(dst_hbm, ones_hbm, zeros_hbm, out_hbm, dst_v, ones_v, deg_sp):
  c = lax.axis_index("c")
  s = lax.axis_index("s")
  wid = c * NS + s
  @pl.when(s == 0)
  def _():
    pltpu.sync_copy(zeros_hbm, deg_sp)
  pltpu.sync_copy(ones_hbm, ones_v)
  pltpu.sync_copy(dst_hbm.at[wid], dst_v)
  plsc.subcore_barrier()

  def chunk(j, carry):
    pltpu.sync_copy(ones_v, deg_sp.at[dst_v.at[j]], add=True)
    return carry
  lax.fori_loop(0, NCH, chunk, 0)

  plsc.subcore_barrier()
  @pl.when(s == 0)
  def _():
    pltpu.sync_copy(deg_sp, out_hbm.at[c])


_deg_call = functools.partial(
    pl.kernel,
    out_type=jax.ShapeDtypeStruct((NC, N), jnp.float32),
    mesh=_mesh,
    scratch_types=[
        pltpu.VMEM((NCH, K), jnp.int32),
        pltpu.VMEM((K,), jnp.float32),
        pltpu.VMEM_SHARED((N,), jnp.float32),
    ],
)(_deg_body)


def _prop_body(g_hbm, src_hbm, dst_hbm, zeros_hbm, out_hbm,
               src_v, dst_v, *rest):
  rows = rest[:NBUF]
  acc_sp = rest[NBUF]
  gsems = rest[NBUF + 1:2 * NBUF + 1]
  ssems = rest[2 * NBUF + 1:]
  c = lax.axis_index("c")
  s = lax.axis_index("s")
  wid = c * NS + s
  pltpu.sync_copy(zeros_hbm.at[pl.ds(s * RPS, RPS)],
                  acc_sp.at[pl.ds(s * RPS, RPS)])
  pltpu.sync_copy(src_hbm.at[wid], src_v)
  pltpu.sync_copy(dst_hbm.at[wid], dst_v)
  plsc.subcore_barrier()

  def gather_start(chunk, b):
    pltpu.async_copy(g_hbm.at[src_v.at[chunk]], rows[b], gsems[b])

  def gather_wait(b):
    pltpu.make_async_copy(g_hbm.at[src_v.at[0]], rows[b], gsems[b]).wait()

  def scatter_start(chunk, b):
    pltpu.async_copy(rows[b], acc_sp.at[dst_v.at[chunk]], ssems[b], add=True)

  def scatter_wait(b):
    pltpu.make_async_copy(rows[b], acc_sp.at[dst_v.at[0]], ssems[b]).wait()

  def round_body(rd, carry):
    base = rd * NBUF
    for b in range(NBUF):
      @pl.when(base + b < NCH)
      def _(b=b):
        gather_start(base + b, b)
    for b in range(NBUF):
      @pl.when(base + b < NCH)
      def _(b=b):
        gather_wait(b)
        scatter_start(base + b, b)
    for b in range(NBUF):
      @pl.when(base + b < NCH)
      def _(b=b):
        scatter_wait(b)
    return carry
  lax.fori_loop(0, ROUNDS, round_body, 0)

  plsc.subcore_barrier()
  pltpu.sync_copy(acc_sp.at[pl.ds(s * RPS, RPS)],
                  out_hbm.at[c, pl.ds(s * RPS, RPS)])


_prop_call = functools.partial(
    pl.kernel,
    out_type=jax.ShapeDtypeStruct((NC, NP, DH), jnp.float32),
    mesh=_mesh,
    compiler_params=pltpu.CompilerParams(use_tc_tiling_on_sc=False),
    scratch_types=(
        [pltpu.VMEM((NCH, K), jnp.int32),
         pltpu.VMEM((NCH, K), jnp.int32)]
        + [pltpu.VMEM((K, DH), jnp.float32) for _ in range(NBUF)]
        + [pltpu.VMEM_SHARED((NP, DH), jnp.float32)]
        + [pltpu.SemaphoreType.DMA for _ in range(2 * NBUF)]
    ),
)(_prop_body)


R = 1000


def _tc_a_body(x_ref, w_ref, degp_ref, ga_ref, gb_ref, dinv_ref):
  d = degp_ref[...]
  deg = 1.0 + d[0] + d[1]
  dinv = lax.rsqrt(deg)
  h = jnp.dot(x_ref[...], w_ref[...], preferred_element_type=jnp.float32)
  g = h * dinv
  ga_ref[...] = g[:, :DH]
  gb_ref[...] = g[:, DH:]
  dinv_ref[...] = dinv


def _tc_a(x, W1, degp):
  return pl.pallas_call(
      _tc_a_body,
      grid=(N // R,),
      in_specs=[
          pl.BlockSpec((R, D), lambda i: (i, 0)),
          pl.BlockSpec((D, D), lambda i: (0, 0)),
          pl.BlockSpec((NC, R, 1), lambda i: (0, i, 0)),
      ],
      out_specs=[
          pl.BlockSpec((R, DH), lambda i: (i, 0)),
          pl.BlockSpec((R, DH), lambda i: (i, 0)),
          pl.BlockSpec((R, 1), lambda i: (i, 0)),
      ],
      out_shape=[
          jax.ShapeDtypeStruct((N, DH), jnp.float32),
          jax.ShapeDtypeStruct((N, DH), jnp.float32),
          jax.ShapeDtypeStruct((N, 1), jnp.float32),
      ],
  )(x, W1, degp)


def _tc_b_body(pa_ref, pb_ref, ga_ref, gb_ref, dinv_ref, b1_ref, w2_ref,
               h1_ref, ga2_ref, gb2_ref):
  pa = pa_ref[...]
  pb = pb_ref[...]
  dinv = dinv_ref[...]
  sa = pa[0] + pa[1] + ga_ref[...]
  sb = pb[0] + pb[1] + gb_ref[...]
  out1 = jnp.concatenate([sa, sb], axis=1) * dinv + b1_ref[...]
  h1 = jnp.maximum(out1, 0.0)
  h1_ref[...] = h1
  h2 = jnp.dot(h1, w2_ref[...], preferred_element_type=jnp.float32)
  g2 = h2 * dinv
  ga2_ref[...] = g2[:, :DH]
  gb2_ref[...] = g2[:, DH:]


def _tc_b(pa, pb, ga, gb, dinv, b1, W2):
  return pl.pallas_call(
      _tc_b_body,
      grid=(N // R,),
      in_specs=[
          pl.BlockSpec((NC, R, DH), lambda i: (0, i, 0)),
          pl.BlockSpec((NC, R, DH), lambda i: (0, i, 0)),
          pl.BlockSpec((R, DH), lambda i: (i, 0)),
          pl.BlockSpec((R, DH), lambda i: (i, 0)),
          pl.BlockSpec((R, 1), lambda i: (i, 0)),
          pl.BlockSpec((1, D), lambda i: (0, 0)),
          pl.BlockSpec((D, D), lambda i: (0, 0)),
      ],
      out_specs=[
          pl.BlockSpec((R, D), lambda i: (i, 0)),
          pl.BlockSpec((R, DH), lambda i: (i, 0)),
          pl.BlockSpec((R, DH), lambda i: (i, 0)),
      ],
      out_shape=[
          jax.ShapeDtypeStruct((N, D), jnp.float32),
          jax.ShapeDtypeStruct((N, DH), jnp.float32),
          jax.ShapeDtypeStruct((N, DH), jnp.float32),
      ],
  )(pa, pb, ga, gb, dinv, b1, W2)


def _tc_c_body(pa_ref, pb_ref, ga_ref, gb_ref, dinv_ref, b2_ref, out_ref):
  pa = pa_ref[...]
  pb = pb_ref[...]
  dinv = dinv_ref[...]
  sa = pa[0] + pa[1] + ga_ref[...]
  sb = pb[0] + pb[1] + gb_ref[...]
  out_ref[...] = jnp.concatenate([sa, sb], axis=1) * dinv + b2_ref[...]


def _tc_c(pa, pb, ga, gb, dinv, b2):
  return pl.pallas_call(
      _tc_c_body,
      grid=(N // R,),
      in_specs=[
          pl.BlockSpec((NC, R, DH), lambda i: (0, i, 0)),
          pl.BlockSpec((NC, R, DH), lambda i: (0, i, 0)),
          pl.BlockSpec((R, DH), lambda i: (i, 0)),
          pl.BlockSpec((R, DH), lambda i: (i, 0)),
          pl.BlockSpec((R, 1), lambda i: (i, 0)),
          pl.BlockSpec((1, D), lambda i: (0, 0)),
      ],
      out_specs=pl.BlockSpec((R, D), lambda i: (i, 0)),
      out_shape=jax.ShapeDtypeStruct((N, D), jnp.float32),
  )(pa, pb, ga, gb, dinv, b2)


@jax.jit
def kernel(x, adj_t, W1, b1, W2, b2):
  src = adj_t[0].astype(jnp.int32).reshape(NW, NCH, K)
  dst = adj_t[1].astype(jnp.int32).reshape(NW, NCH, K)
  zeros_hd = jnp.zeros((NP, DH), jnp.float32)
  zeros_n = jnp.zeros((N,), jnp.float32)
  ones_k = jnp.ones((K,), jnp.float32)

  degp = _deg_call(dst, ones_k, zeros_n)
  ga1, gb1, dinv = _tc_a(x, W1, degp.reshape(NC, N, 1))
  pa1 = _prop_call(ga1, src, dst, zeros_hd)
  pb1 = _prop_call(gb1, src, dst, zeros_hd)
  h1, ga2, gb2 = _tc_b(pa1, pb1, ga1, gb1, dinv, b1.reshape(1, D), W2)
  pa2 = _prop_call(ga2, src, dst, zeros_hd)
  pb2 = _prop_call(gb2, src, dst, zeros_hd)
  logit = _tc_c(pa2, pb2, ga2, gb2, dinv, b2.reshape(1, D))
  return (logit, h1)


# speedup vs baseline: 1.7717x; 1.0709x over previous
import functools

import jax
import jax.numpy as jnp
from jax import lax
from jax.experimental import pallas as pl
from jax.experimental.pallas import tpu as pltpu
from jax.experimental.pallas import tpu_sc as plsc

N = 10000
E = 320000
D = 128
DH = D // 2
NC = 2
NS = 16
NW = NC * NS
EPT = E // NW
K = 80
NCH = EPT // K
NP = 10112
RPS = NP // NS
NBUF = 12
ROUNDS = -(-NCH // NBUF)

_mesh = plsc.VectorSubcoreMesh(core_axis_name="c", subcore_axis_name="s")


def _deg_body(dst_hbm, ones_hbm, zeros_hbm, out_hbm, dst_v, ones_v, deg_sp):
  c = lax.axis_index("c")
  s = lax.axis_index("s")
  wid = c * NS + s
  @pl.when(s == 0)
  def _():
    pltpu.sync_copy(zeros_hbm, deg_sp)
  pltpu.sync_copy(ones_hbm, ones_v)
  pltpu.sync_copy(dst_hbm.at[wid], dst_v)
  plsc.subcore_barrier()

  def chunk(j, carry):
    pltpu.sync_copy(ones_v, deg_sp.at[dst_v.at[j]], add=True)
    return carry
  lax.fori_loop(0, NCH, chunk, 0)

  plsc.subcore_barrier()
  @pl.when(s == 0)
  def _():
    pltpu.sync_copy(deg_sp, out_hbm.at[c])


_deg_call = functools.partial(
    pl.kernel,
    out_type=jax.ShapeDtypeStruct((NC, N), jnp.float32),
    mesh=_mesh,
    scratch_types=[
        pltpu.VMEM((NCH, K), jnp.int32),
        pltpu.VMEM((K,), jnp.float32),
        pltpu.VMEM_SHARED((N,), jnp.float32),
    ],
)(_deg_body)


def _prop_body(g_hbm, src_hbm, dst_hbm, zeros_hbm, out_hbm,
               src_v, dst_v, *rest):
  rows = rest[:NBUF]
  acc_sp = rest[NBUF]
  gsems = rest[NBUF + 1:2 * NBUF + 1]
  ssems = rest[2 * NBUF + 1:]
  c = lax.axis_index("c")
  s = lax.axis_index("s")
  wid = c * NS + s
  pltpu.sync_copy(zeros_hbm.at[pl.ds(s * RPS, RPS)],
                  acc_sp.at[pl.ds(s * RPS, RPS)])
  pltpu.sync_copy(src_hbm.at[wid], src_v)
  pltpu.sync_copy(dst_hbm.at[wid], dst_v)
  plsc.subcore_barrier()

  def gather_start(chunk, b):
    pltpu.async_copy(g_hbm.at[src_v.at[chunk]], rows[b], gsems[b])

  def gather_wait(b):
    pltpu.make_async_copy(g_hbm.at[src_v.at[0]], rows[b], gsems[b]).wait()

  def scatter_start(chunk, b):
    pltpu.async_copy(rows[b], acc_sp.at[dst_v.at[chunk]], ssems[b], add=True)

  def scatter_wait(b):
    pltpu.make_async_copy(rows[b], acc_sp.at[dst_v.at[0]], ssems[b]).wait()

  def round_body(rd, carry):
    base = rd * NBUF
    for b in range(NBUF):
      @pl.when(base + b < NCH)
      def _(b=b):
        gather_start(base + b, b)
    for b in range(NBUF):
      @pl.when(base + b < NCH)
      def _(b=b):
        gather_wait(b)
        scatter_start(base + b, b)
    for b in range(NBUF):
      @pl.when(base + b < NCH)
      def _(b=b):
        scatter_wait(b)
    return carry
  lax.fori_loop(0, ROUNDS, round_body, 0)

  plsc.subcore_barrier()
  pltpu.sync_copy(acc_sp.at[pl.ds(s * RPS, RPS)],
                  out_hbm.at[c, pl.ds(s * RPS, RPS)])


_prop_call = functools.partial(
    pl.kernel,
    out_type=jax.ShapeDtypeStruct((NC, NP, DH), jnp.float32),
    mesh=_mesh,
    compiler_params=pltpu.CompilerParams(use_tc_tiling_on_sc=False),
    scratch_types=(
        [pltpu.VMEM((NCH, K), jnp.int32),
         pltpu.VMEM((NCH, K), jnp.int32)]
        + [pltpu.VMEM((K, DH), jnp.float32) for _ in range(NBUF)]
        + [pltpu.VMEM_SHARED((NP, DH), jnp.float32)]
        + [pltpu.SemaphoreType.DMA for _ in range(2 * NBUF)]
    ),
)(_prop_body)


R = 1000


def _tc_a_body(x_ref, w_ref, degp_ref, ga_ref, gb_ref, dinv_ref):
  d = degp_ref[...]
  deg = 1.0 + d[0] + d[1]
  dinv = lax.rsqrt(deg)
  h = jnp.dot(x_ref[...], w_ref[...], preferred_element_type=jnp.float32)
  g = h * dinv
  ga_ref[...] = g[:, :DH]
  gb_ref[...] = g[:, DH:]
  dinv_ref[...] = dinv


def _tc_a(x, W1, degp):
  return pl.pallas_call(
      _tc_a_body,
      grid=(N // R,),
      in_specs=[
          pl.BlockSpec((R, D), lambda i: (i, 0)),
          pl.BlockSpec((D, D), lambda i: (0, 0)),
          pl.BlockSpec((NC, R, 1), lambda i: (0, i, 0)),
      ],
      out_specs=[
          pl.BlockSpec((R, DH), lambda i: (i, 0)),
          pl.BlockSpec((R, DH), lambda i: (i, 0)),
          pl.BlockSpec((R, 1), lambda i: (i, 0)),
      ],
      out_shape=[
          jax.ShapeDtypeStruct((N, DH), jnp.float32),
          jax.ShapeDtypeStruct((N, DH), jnp.float32),
          jax.ShapeDtypeStruct((N, 1), jnp.float32),
      ],
  )(x, W1, degp)


def _tc_b_body(pa_ref, pb_ref, ga_ref, gb_ref, dinv_ref, b1_ref, w2_ref,
               h1_ref, ga2_ref, gb2_ref):
  pa = pa_ref[...]
  pb = pb_ref[...]
  dinv = dinv_ref[...]
  sa = pa[0] + pa[1] + ga_ref[...]
  sb = pb[0] + pb[1] + gb_ref[...]
  out1 = jnp.concatenate([sa, sb], axis=1) * dinv + b1_ref[...]
  h1 = jnp.maximum(out1, 0.0)
  h1_ref[...] = h1
  h2 = jnp.dot(h1, w2_ref[...], preferred_element_type=jnp.float32)
  g2 = h2 * dinv
  ga2_ref[...] = g2[:, :DH]
  gb2_ref[...] = g2[:, DH:]


def _tc_b(pa, pb, ga, gb, dinv, b1, W2):
  return pl.pallas_call(
      _tc_b_body,
      grid=(N // R,),
      in_specs=[
          pl.BlockSpec((NC, R, DH), lambda i: (0, i, 0)),
          pl.BlockSpec((NC, R, DH), lambda i: (0, i, 0)),
          pl.BlockSpec((R, DH), lambda i: (i, 0)),
          pl.BlockSpec((R, DH), lambda i: (i, 0)),
          pl.BlockSpec((R, 1), lambda i: (i, 0)),
          pl.BlockSpec((1, D), lambda i: (0, 0)),
          pl.BlockSpec((D, D), lambda i: (0, 0)),
      ],
      out_specs=[
          pl.BlockSpec((R, D), lambda i: (i, 0)),
          pl.BlockSpec((R, DH), lambda i: (i, 0)),
          pl.BlockSpec((R, DH), lambda i: (i, 0)),
      ],
      out_shape=[
          jax.ShapeDtypeStruct((N, D), jnp.float32),
          jax.ShapeDtypeStruct((N, DH), jnp.float32),
          jax.ShapeDtypeStruct((N, DH), jnp.float32),
      ],
  )(pa, pb, ga, gb, dinv, b1, W2)


def _tc_c_body(pa_ref, pb_ref, ga_ref, gb_ref, dinv_ref, b2_ref, out_ref):
  pa = pa_ref[...]
  pb = pb_ref[...]
  dinv = dinv_ref[...]
  sa = pa[0] + pa[1] + ga_ref[...]
  sb = pb[0] + pb[1] + gb_ref[...]
  out_ref[...] = jnp.concatenate([sa, sb], axis=1) * dinv + b2_ref[...]


def _tc_c(pa, pb, ga, gb, dinv, b2):
  return pl.pallas_call(
      _tc_c_body,
      grid=(N // R,),
      in_specs=[
          pl.BlockSpec((NC, R, DH), lambda i: (0, i, 0)),
          pl.BlockSpec((NC, R, DH), lambda i: (0, i, 0)),
          pl.BlockSpec((R, DH), lambda i: (i, 0)),
          pl.BlockSpec((R, DH), lambda i: (i, 0)),
          pl.BlockSpec((R, 1), lambda i: (i, 0)),
          pl.BlockSpec((1, D), lambda i: (0, 0)),
      ],
      out_specs=pl.BlockSpec((R, D), lambda i: (i, 0)),
      out_shape=jax.ShapeDtypeStruct((N, D), jnp.float32),
  )(pa, pb, ga, gb, dinv, b2)


@jax.jit
def kernel(x, adj_t, W1, b1, W2, b2):
  src = adj_t[0].astype(jnp.int32).reshape(NW, NCH, K)
  dst = adj_t[1].astype(jnp.int32).reshape(NW, NCH, K)
  zeros_hd = jnp.zeros((NP, DH), jnp.float32)
  zeros_n = jnp.zeros((N,), jnp.float32)
  ones_k = jnp.ones((K,), jnp.float32)

  degp = _deg_call(dst, ones_k, zeros_n)
  ga1, gb1, dinv = _tc_a(x, W1, degp.reshape(NC, N, 1))
  pa1 = _prop_call(ga1, src, dst, zeros_hd)
  pb1 = _prop_call(gb1, src, dst, zeros_hd)
  h1, ga2, gb2 = _tc_b(pa1, pb1, ga1, gb1, dinv, b1.reshape(1, D), W2)
  pa2 = _prop_call(ga2, src, dst, zeros_hd)
  pb2 = _prop_call(gb2, src, dst, zeros_hd)
  logit = _tc_c(pa2, pb2, ga2, gb2, dinv, b2.reshape(1, D))
  return (logit, h1)


# speedup vs baseline: 1.7778x; 1.0034x over previous
import functools

import jax
import jax.numpy as jnp
from jax import lax
from jax.experimental import pallas as pl
from jax.experimental.pallas import tpu as pltpu
from jax.experimental.pallas import tpu_sc as plsc

N = 10000
E = 320000
D = 128
DH = D // 2
NC = 2
NS = 16
NW = NC * NS
EPT = E // NW
K = 80
NCH = EPT // K
NP = 10112
RPS = NP // NS
NBUF = 12
ROUNDS = -(-NCH // NBUF)

_mesh = plsc.VectorSubcoreMesh(core_axis_name="c", subcore_axis_name="s")


def _deg_body(dst_hbm, ones_hbm, zeros_hbm, out_hbm, dst_v, ones_v, deg_sp):
  c = lax.axis_index("c")
  s = lax.axis_index("s")
  wid = c * NS + s
  @pl.when(s == 0)
  def _():
    pltpu.sync_copy(zeros_hbm, deg_sp)
  pltpu.sync_copy(ones_hbm, ones_v)
  pltpu.sync_copy(dst_hbm.at[wid], dst_v)
  plsc.subcore_barrier()

  def chunk(j, carry):
    pltpu.sync_copy(ones_v, deg_sp.at[dst_v.at[j]], add=True)
    return carry
  lax.fori_loop(0, NCH, chunk, 0)

  plsc.subcore_barrier()
  @pl.when(s == 0)
  def _():
    pltpu.sync_copy(deg_sp, out_hbm.at[c])


_deg_call = functools.partial(
    pl.kernel,
    out_type=jax.ShapeDtypeStruct((NC, N), jnp.float32),
    mesh=_mesh,
    scratch_types=[
        pltpu.VMEM((NCH, K), jnp.int32),
        pltpu.VMEM((K,), jnp.float32),
        pltpu.VMEM_SHARED((N,), jnp.float32),
    ],
)(_deg_body)


def _prop_body(ga_hbm, gb_hbm, src_hbm, dst_hbm, zeros_hbm,
               outa_hbm, outb_hbm, src_v, dst_v, *rest):
  rows = rest[:NBUF]
  acc_sp = rest[NBUF]
  gsems = rest[NBUF + 1:2 * NBUF + 1]
  ssems = rest[2 * NBUF + 1:]
  c = lax.axis_index("c")
  s = lax.axis_index("s")
  wid = c * NS + s
  pltpu.sync_copy(src_hbm.at[wid], src_v)
  pltpu.sync_copy(dst_hbm.at[wid], dst_v)

  for g_hbm, out_hbm in ((ga_hbm, outa_hbm), (gb_hbm, outb_hbm)):
    pltpu.sync_copy(zeros_hbm.at[pl.ds(s * RPS, RPS)],
                    acc_sp.at[pl.ds(s * RPS, RPS)])
    plsc.subcore_barrier()

    def gather_start(chunk, b):
      pltpu.async_copy(g_hbm.at[src_v.at[chunk]], rows[b], gsems[b])

    def gather_wait(b):
      pltpu.make_async_copy(g_hbm.at[src_v.at[0]], rows[b], gsems[b]).wait()

    def scatter_start(chunk, b):
      pltpu.async_copy(rows[b], acc_sp.at[dst_v.at[chunk]], ssems[b],
                       add=True)

    def scatter_wait(b):
      pltpu.make_async_copy(rows[b], acc_sp.at[dst_v.at[0]],
                            ssems[b]).wait()

    def round_body(rd, carry):
      base = rd * NBUF
      for b in range(NBUF):
        @pl.when(base + b < NCH)
        def _(b=b):
          gather_start(base + b, b)
      for b in range(NBUF):
        @pl.when(base + b < NCH)
        def _(b=b):
          gather_wait(b)
          scatter_start(base + b, b)
      for b in range(NBUF):
        @pl.when(base + b < NCH)
        def _(b=b):
          scatter_wait(b)
      return carry
    lax.fori_loop(0, ROUNDS, round_body, 0)

    plsc.subcore_barrier()
    pltpu.sync_copy(acc_sp.at[pl.ds(s * RPS, RPS)],
                    out_hbm.at[c, pl.ds(s * RPS, RPS)])
    plsc.subcore_barrier()


_prop_call = functools.partial(
    pl.kernel,
    out_type=[
        jax.ShapeDtypeStruct((NC, NP, DH), jnp.float32),
        jax.ShapeDtypeStruct((NC, NP, DH), jnp.float32),
    ],
    mesh=_mesh,
    compiler_params=pltpu.CompilerParams(use_tc_tiling_on_sc=False),
    scratch_types=(
        [pltpu.VMEM((NCH, K), jnp.int32),
         pltpu.VMEM((NCH, K), jnp.int32)]
        + [pltpu.VMEM((K, DH), jnp.float32) for _ in range(NBUF)]
        + [pltpu.VMEM_SHARED((NP, DH), jnp.float32)]
        + [pltpu.SemaphoreType.DMA for _ in range(2 * NBUF)]
    ),
)(_prop_body)


R = 1000


def _tc_a_body(x_ref, w_ref, degp_ref, ga_ref, gb_ref, dinv_ref):
  d = degp_ref[...]
  deg = 1.0 + d[0] + d[1]
  dinv = lax.rsqrt(deg)
  h = jnp.dot(x_ref[...], w_ref[...], preferred_element_type=jnp.float32)
  g = h * dinv
  ga_ref[...] = g[:, :DH]
  gb_ref[...] = g[:, DH:]
  dinv_ref[...] = dinv


def _tc_a(x, W1, degp):
  return pl.pallas_call(
      _tc_a_body,
      grid=(N // R,),
      in_specs=[
          pl.BlockSpec((R, D), lambda i: (i, 0)),
          pl.BlockSpec((D, D), lambda i: (0, 0)),
          pl.BlockSpec((NC, R, 1), lambda i: (0, i, 0)),
      ],
      out_specs=[
          pl.BlockSpec((R, DH), lambda i: (i, 0)),
          pl.BlockSpec((R, DH), lambda i: (i, 0)),
          pl.BlockSpec((R, 1), lambda i: (i, 0)),
      ],
      out_shape=[
          jax.ShapeDtypeStruct((N, DH), jnp.float32),
          jax.ShapeDtypeStruct((N, DH), jnp.float32),
          jax.ShapeDtypeStruct((N, 1), jnp.float32),
      ],
  )(x, W1, degp)


def _tc_b_body(pa_ref, pb_ref, ga_ref, gb_ref, dinv_ref, b1_ref, w2_ref,
               h1_ref, ga2_ref, gb2_ref):
  pa = pa_ref[...]
  pb = pb_ref[...]
  dinv = dinv_ref[...]
  sa = pa[0] + pa[1] + ga_ref[...]
  sb = pb[0] + pb[1] + gb_ref[...]
  out1 = jnp.concatenate([sa, sb], axis=1) * dinv + b1_ref[...]
  h1 = jnp.maximum(out1, 0.0)
  h1_ref[...] = h1
  h2 = jnp.dot(h1, w2_ref[...], preferred_element_type=jnp.float32)
  g2 = h2 * dinv
  ga2_ref[...] = g2[:, :DH]
  gb2_ref[...] = g2[:, DH:]


def _tc_b(pa, pb, ga, gb, dinv, b1, W2):
  return pl.pallas_call(
      _tc_b_body,
      grid=(N // R,),
      in_specs=[
          pl.BlockSpec((NC, R, DH), lambda i: (0, i, 0)),
          pl.BlockSpec((NC, R, DH), lambda i: (0, i, 0)),
          pl.BlockSpec((R, DH), lambda i: (i, 0)),
          pl.BlockSpec((R, DH), lambda i: (i, 0)),
          pl.BlockSpec((R, 1), lambda i: (i, 0)),
          pl.BlockSpec((1, D), lambda i: (0, 0)),
          pl.BlockSpec((D, D), lambda i: (0, 0)),
      ],
      out_specs=[
          pl.BlockSpec((R, D), lambda i: (i, 0)),
          pl.BlockSpec((R, DH), lambda i: (i, 0)),
          pl.BlockSpec((R, DH), lambda i: (i, 0)),
      ],
      out_shape=[
          jax.ShapeDtypeStruct((N, D), jnp.float32),
          jax.ShapeDtypeStruct((N, DH), jnp.float32),
          jax.ShapeDtypeStruct((N, DH), jnp.float32),
      ],
  )(pa, pb, ga, gb, dinv, b1, W2)


def _tc_c_body(pa_ref, pb_ref, ga_ref, gb_ref, dinv_ref, b2_ref, out_ref):
  pa = pa_ref[...]
  pb = pb_ref[...]
  dinv = dinv_ref[...]
  sa = pa[0] + pa[1] + ga_ref[...]
  sb = pb[0] + pb[1] + gb_ref[...]
  out_ref[...] = jnp.concatenate([sa, sb], axis=1) * dinv + b2_ref[...]


def _tc_c(pa, pb, ga, gb, dinv, b2):
  return pl.pallas_call(
      _tc_c_body,
      grid=(N // R,),
      in_specs=[
          pl.BlockSpec((NC, R, DH), lambda i: (0, i, 0)),
          pl.BlockSpec((NC, R, DH), lambda i: (0, i, 0)),
          pl.BlockSpec((R, DH), lambda i: (i, 0)),
          pl.BlockSpec((R, DH), lambda i: (i, 0)),
          pl.BlockSpec((R, 1), lambda i: (i, 0)),
          pl.BlockSpec((1, D), lambda i: (0, 0)),
      ],
      out_specs=pl.BlockSpec((R, D), lambda i: (i, 0)),
      out_shape=jax.ShapeDtypeStruct((N, D), jnp.float32),
  )(pa, pb, ga, gb, dinv, b2)


@jax.jit
def kernel(x, adj_t, W1, b1, W2, b2):
  src = adj_t[0].astype(jnp.int32).reshape(NW, NCH, K)
  dst = adj_t[1].astype(jnp.int32).reshape(NW, NCH, K)
  zeros_hd = jnp.zeros((NP, DH), jnp.float32)
  zeros_n = jnp.zeros((N,), jnp.float32)
  ones_k = jnp.ones((K,), jnp.float32)

  degp = _deg_call(dst, ones_k, zeros_n)
  ga1, gb1, dinv = _tc_a(x, W1, degp.reshape(NC, N, 1))
  pa1, pb1 = _prop_call(ga1, gb1, src, dst, zeros_hd)
  h1, ga2, gb2 = _tc_b(pa1, pb1, ga1, gb1, dinv, b1.reshape(1, D), W2)
  pa2, pb2 = _prop_call(ga2, gb2, src, dst, zeros_hd)
  logit = _tc_c(pa2, pb2, ga2, gb2, dinv, b2.reshape(1, D))
  return (logit, h1)


# speedup vs baseline: 1.7934x; 1.0088x over previous
import functools

import jax
import jax.numpy as jnp
from jax import lax
from jax.experimental import pallas as pl
from jax.experimental.pallas import tpu as pltpu
from jax.experimental.pallas import tpu_sc as plsc

N = 10000
E = 320000
D = 128
DH = D // 2
NC = 2
NS = 16
NW = NC * NS
EPT = E // NW
K = 80
NCH = EPT // K
NP = 10112
RPS = NP // NS
NBUF = 12
ROUNDS = -(-NCH // NBUF)

_mesh = plsc.VectorSubcoreMesh(core_axis_name="c", subcore_axis_name="s")


def _deg_body(dst_hbm, ones_hbm, zeros_hbm, out_hbm, dst_v, ones_v, deg_sp):
  c = lax.axis_index("c")
  s = lax.axis_index("s")
  wid = c * NS + s
  @pl.when(s == 0)
  def _():
    pltpu.sync_copy(zeros_hbm, deg_sp)
  pltpu.sync_copy(ones_hbm, ones_v)
  pltpu.sync_copy(dst_hbm.at[wid], dst_v)
  plsc.subcore_barrier()

  def chunk(j, carry):
    pltpu.sync_copy(ones_v, deg_sp.at[dst_v.at[j]], add=True)
    return carry
  lax.fori_loop(0, NCH, chunk, 0)

  plsc.subcore_barrier()
  @pl.when(s == 0)
  def _():
    pltpu.sync_copy(deg_sp, out_hbm.at[c])


_deg_call = functools.partial(
    pl.kernel,
    out_type=jax.ShapeDtypeStruct((NC, N), jnp.float32),
    mesh=_mesh,
    scratch_types=[
        pltpu.VMEM((NCH, K), jnp.int32),
        pltpu.VMEM((K,), jnp.float32),
        pltpu.VMEM_SHARED((N,), jnp.float32),
    ],
)(_deg_body)


def _prop_body(ga_hbm, gb_hbm, src_hbm, dst_hbm, zeros_hbm,
               outa_hbm, outb_hbm, src_v, dst_v, *rest):
  rows = rest[:NBUF]
  acc_sp = rest[NBUF]
  gsems = rest[NBUF + 1:2 * NBUF + 1]
  ssems = rest[2 * NBUF + 1:]
  c = lax.axis_index("c")
  s = lax.axis_index("s")
  wid = c * NS + s
  pltpu.sync_copy(src_hbm.at[wid], src_v)
  pltpu.sync_copy(dst_hbm.at[wid], dst_v)

  for g_hbm, out_hbm in ((ga_hbm, outa_hbm), (gb_hbm, outb_hbm)):
    pltpu.sync_copy(zeros_hbm.at[pl.ds(s * RPS, RPS)],
                    acc_sp.at[pl.ds(s * RPS, RPS)])
    plsc.subcore_barrier()

    def gather_start(chunk, b):
      pltpu.async_copy(g_hbm.at[src_v.at[chunk]], rows[b], gsems[b])

    def gather_wait(b):
      pltpu.make_async_copy(g_hbm.at[src_v.at[0]], rows[b], gsems[b]).wait()

    def scatter_start(chunk, b):
      pltpu.async_copy(rows[b], acc_sp.at[dst_v.at[chunk]], ssems[b],
                       add=True)

    def scatter_wait(b):
      pltpu.make_async_copy(rows[b], acc_sp.at[dst_v.at[0]],
                            ssems[b]).wait()

    def round_body(rd, carry):
      base = rd * NBUF
      for b in range(NBUF):
        @pl.when(base + b < NCH)
        def _(b=b):
          gather_start(base + b, b)
      for b in range(NBUF):
        @pl.when(base + b < NCH)
        def _(b=b):
          gather_wait(b)
          scatter_start(base + b, b)
      for b in range(NBUF):
        @pl.when(base + b < NCH)
        def _(b=b):
          scatter_wait(b)
      return carry
    lax.fori_loop(0, ROUNDS, round_body, 0)

    plsc.subcore_barrier()
    pltpu.sync_copy(acc_sp.at[pl.ds(s * RPS, RPS)],
                    out_hbm.at[c, pl.ds(s * RPS, RPS)])
    plsc.subcore_barrier()


_prop_call = functools.partial(
    pl.kernel,
    out_type=[
        jax.ShapeDtypeStruct((NC, NP, DH), jnp.float32),
        jax.ShapeDtypeStruct((NC, NP, DH), jnp.float32),
    ],
    mesh=_mesh,
    compiler_params=pltpu.CompilerParams(use_tc_tiling_on_sc=False),
    scratch_types=(
        [pltpu.VMEM((NCH, K), jnp.int32),
         pltpu.VMEM((NCH, K), jnp.int32)]
        + [pltpu.VMEM((K, DH), jnp.float32) for _ in range(NBUF)]
        + [pltpu.VMEM_SHARED((NP, DH), jnp.float32)]
        + [pltpu.SemaphoreType.DMA for _ in range(2 * NBUF)]
    ),
)(_prop_body)


R = 2000


def _tc_a_body(x_ref, w_ref, degp_ref, ga_ref, gb_ref, dinv_ref):
  d = degp_ref[...]
  deg = 1.0 + d[0] + d[1]
  dinv = lax.rsqrt(deg)
  h = jnp.dot(x_ref[...], w_ref[...], preferred_element_type=jnp.float32)
  g = h * dinv
  ga_ref[...] = g[:, :DH]
  gb_ref[...] = g[:, DH:]
  dinv_ref[...] = dinv


def _tc_a(x, W1, degp):
  return pl.pallas_call(
      _tc_a_body,
      grid=(N // R,),
      in_specs=[
          pl.BlockSpec((R, D), lambda i: (i, 0)),
          pl.BlockSpec((D, D), lambda i: (0, 0)),
          pl.BlockSpec((NC, R, 1), lambda i: (0, i, 0)),
      ],
      out_specs=[
          pl.BlockSpec((R, DH), lambda i: (i, 0)),
          pl.BlockSpec((R, DH), lambda i: (i, 0)),
          pl.BlockSpec((R, 1), lambda i: (i, 0)),
      ],
      out_shape=[
          jax.ShapeDtypeStruct((N, DH), jnp.float32),
          jax.ShapeDtypeStruct((N, DH), jnp.float32),
          jax.ShapeDtypeStruct((N, 1), jnp.float32),
      ],
  )(x, W1, degp)


def _tc_b_body(pa_ref, pb_ref, ga_ref, gb_ref, dinv_ref, b1_ref, w2_ref,
               h1_ref, ga2_ref, gb2_ref):
  pa = pa_ref[...]
  pb = pb_ref[...]
  dinv = dinv_ref[...]
  sa = pa[0] + pa[1] + ga_ref[...]
  sb = pb[0] + pb[1] + gb_ref[...]
  out1 = jnp.concatenate([sa, sb], axis=1) * dinv + b1_ref[...]
  h1 = jnp.maximum(out1, 0.0)
  h1_ref[...] = h1
  h2 = jnp.dot(h1, w2_ref[...], preferred_element_type=jnp.float32)
  g2 = h2 * dinv
  ga2_ref[...] = g2[:, :DH]
  gb2_ref[...] = g2[:, DH:]


def _tc_b(pa, pb, ga, gb, dinv, b1, W2):
  return pl.pallas_call(
      _tc_b_body,
      grid=(N // R,),
      in_specs=[
          pl.BlockSpec((NC, R, DH), lambda i: (0, i, 0)),
          pl.BlockSpec((NC, R, DH), lambda i: (0, i, 0)),
          pl.BlockSpec((R, DH), lambda i: (i, 0)),
          pl.BlockSpec((R, DH), lambda i: (i, 0)),
          pl.BlockSpec((R, 1), lambda i: (i, 0)),
          pl.BlockSpec((1, D), lambda i: (0, 0)),
          pl.BlockSpec((D, D), lambda i: (0, 0)),
      ],
      out_specs=[
          pl.BlockSpec((R, D), lambda i: (i, 0)),
          pl.BlockSpec((R, DH), lambda i: (i, 0)),
          pl.BlockSpec((R, DH), lambda i: (i, 0)),
      ],
      out_shape=[
          jax.ShapeDtypeStruct((N, D), jnp.float32),
          jax.ShapeDtypeStruct((N, DH), jnp.float32),
          jax.ShapeDtypeStruct((N, DH), jnp.float32),
      ],
  )(pa, pb, ga, gb, dinv, b1, W2)


def _tc_c_body(pa_ref, pb_ref, ga_ref, gb_ref, dinv_ref, b2_ref, out_ref):
  pa = pa_ref[...]
  pb = pb_ref[...]
  dinv = dinv_ref[...]
  sa = pa[0] + pa[1] + ga_ref[...]
  sb = pb[0] + pb[1] + gb_ref[...]
  out_ref[...] = jnp.concatenate([sa, sb], axis=1) * dinv + b2_ref[...]


def _tc_c(pa, pb, ga, gb, dinv, b2):
  return pl.pallas_call(
      _tc_c_body,
      grid=(N // R,),
      in_specs=[
          pl.BlockSpec((NC, R, DH), lambda i: (0, i, 0)),
          pl.BlockSpec((NC, R, DH), lambda i: (0, i, 0)),
          pl.BlockSpec((R, DH), lambda i: (i, 0)),
          pl.BlockSpec((R, DH), lambda i: (i, 0)),
          pl.BlockSpec((R, 1), lambda i: (i, 0)),
          pl.BlockSpec((1, D), lambda i: (0, 0)),
      ],
      out_specs=pl.BlockSpec((R, D), lambda i: (i, 0)),
      out_shape=jax.ShapeDtypeStruct((N, D), jnp.float32),
  )(pa, pb, ga, gb, dinv, b2)


@jax.jit
def kernel(x, adj_t, W1, b1, W2, b2):
  src = adj_t[0].astype(jnp.int32).reshape(NW, NCH, K)
  dst = adj_t[1].astype(jnp.int32).reshape(NW, NCH, K)
  zeros_hd = jnp.zeros((NP, DH), jnp.float32)
  zeros_n = jnp.zeros((N,), jnp.float32)
  ones_k = jnp.ones((K,), jnp.float32)

  degp = _deg_call(dst, ones_k, zeros_n)
  ga1, gb1, dinv = _tc_a(x, W1, degp.reshape(NC, N, 1))
  pa1, pb1 = _prop_call(ga1, gb1, src, dst, zeros_hd)
  h1, ga2, gb2 = _tc_b(pa1, pb1, ga1, gb1, dinv, b1.reshape(1, D), W2)
  pa2, pb2 = _prop_call(ga2, gb2, src, dst, zeros_hd)
  logit = _tc_c(pa2, pb2, ga2, gb2, dinv, b2.reshape(1, D))
  return (logit, h1)


# speedup vs baseline: 1.7987x; 1.0030x over previous
import functools

import jax
import jax.numpy as jnp
from jax import lax
from jax.experimental import pallas as pl
from jax.experimental.pallas import tpu as pltpu
from jax.experimental.pallas import tpu_sc as plsc

N = 10000
E = 320000
D = 128
DH = D // 2
NC = 2
NS = 16
NW = NC * NS
EPT = E // NW
K = 80
NCH = EPT // K
NP = 10112
RPS = NP // NS
NBUF = 13
ROUNDS = -(-NCH // NBUF)

_mesh = plsc.VectorSubcoreMesh(core_axis_name="c", subcore_axis_name="s")


def _deg_body(dst_hbm, ones_hbm, zeros_hbm, out_hbm, dst_v, ones_v, deg_sp):
  c = lax.axis_index("c")
  s = lax.axis_index("s")
  wid = c * NS + s
  @pl.when(s == 0)
  def _():
    pltpu.sync_copy(zeros_hbm, deg_sp)
  pltpu.sync_copy(ones_hbm, ones_v)
  pltpu.sync_copy(dst_hbm.at[wid], dst_v)
  plsc.subcore_barrier()

  def chunk(j, carry):
    pltpu.sync_copy(ones_v, deg_sp.at[dst_v.at[j]], add=True)
    return carry
  lax.fori_loop(0, NCH, chunk, 0)

  plsc.subcore_barrier()
  @pl.when(s == 0)
  def _():
    pltpu.sync_copy(deg_sp, out_hbm.at[c])


_deg_call = functools.partial(
    pl.kernel,
    out_type=jax.ShapeDtypeStruct((NC, N), jnp.float32),
    mesh=_mesh,
    scratch_types=[
        pltpu.VMEM((NCH, K), jnp.int32),
        pltpu.VMEM((K,), jnp.float32),
        pltpu.VMEM_SHARED((N,), jnp.float32),
    ],
)(_deg_body)


def _prop_body(ga_hbm, gb_hbm, src_hbm, dst_hbm, zeros_hbm,
               outa_hbm, outb_hbm, src_v, dst_v, *rest):
  rows = rest[:NBUF]
  acc_sp = rest[NBUF]
  gsems = rest[NBUF + 1:2 * NBUF + 1]
  ssems = rest[2 * NBUF + 1:]
  c = lax.axis_index("c")
  s = lax.axis_index("s")
  wid = c * NS + s
  pltpu.sync_copy(src_hbm.at[wid], src_v)
  pltpu.sync_copy(dst_hbm.at[wid], dst_v)

  for g_hbm, out_hbm in ((ga_hbm, outa_hbm), (gb_hbm, outb_hbm)):
    pltpu.sync_copy(zeros_hbm.at[pl.ds(s * RPS, RPS)],
                    acc_sp.at[pl.ds(s * RPS, RPS)])
    plsc.subcore_barrier()

    def gather_start(chunk, b):
      pltpu.async_copy(g_hbm.at[src_v.at[chunk]], rows[b], gsems[b])

    def gather_wait(b):
      pltpu.make_async_copy(g_hbm.at[src_v.at[0]], rows[b], gsems[b]).wait()

    def scatter_start(chunk, b):
      pltpu.async_copy(rows[b], acc_sp.at[dst_v.at[chunk]], ssems[b],
                       add=True)

    def scatter_wait(b):
      pltpu.make_async_copy(rows[b], acc_sp.at[dst_v.at[0]],
                            ssems[b]).wait()

    def round_body(rd, carry):
      base = rd * NBUF
      for b in range(NBUF):
        @pl.when(base + b < NCH)
        def _(b=b):
          gather_start(base + b, b)
      for b in range(NBUF):
        @pl.when(base + b < NCH)
        def _(b=b):
          gather_wait(b)
          scatter_start(base + b, b)
      for b in range(NBUF):
        @pl.when(base + b < NCH)
        def _(b=b):
          scatter_wait(b)
      return carry
    lax.fori_loop(0, ROUNDS, round_body, 0)

    plsc.subcore_barrier()
    pltpu.sync_copy(acc_sp.at[pl.ds(s * RPS, RPS)],
                    out_hbm.at[c, pl.ds(s * RPS, RPS)])
    plsc.subcore_barrier()


_prop_call = functools.partial(
    pl.kernel,
    out_type=[
        jax.ShapeDtypeStruct((NC, NP, DH), jnp.float32),
        jax.ShapeDtypeStruct((NC, NP, DH), jnp.float32),
    ],
    mesh=_mesh,
    compiler_params=pltpu.CompilerParams(use_tc_tiling_on_sc=False),
    scratch_types=(
        [pltpu.VMEM((NCH, K), jnp.int32),
         pltpu.VMEM((NCH, K), jnp.int32)]
        + [pltpu.VMEM((K, DH), jnp.float32) for _ in range(NBUF)]
        + [pltpu.VMEM_SHARED((NP, DH), jnp.float32)]
        + [pltpu.SemaphoreType.DMA for _ in range(2 * NBUF)]
    ),
)(_prop_body)


R = 2000


def _tc_a_body(x_ref, w_ref, degp_ref, ga_ref, gb_ref, dinv_ref):
  d = degp_ref[...]
  deg = 1.0 + d[0] + d[1]
  dinv = lax.rsqrt(deg)
  h = jnp.dot(x_ref[...], w_ref[...], preferred_element_type=jnp.float32)
  g = h * dinv
  ga_ref[...] = g[:, :DH]
  gb_ref[...] = g[:, DH:]
  dinv_ref[...] = dinv


def _tc_a(x, W1, degp):
  return pl.pallas_call(
      _tc_a_body,
      grid=(N // R,),
      in_specs=[
          pl.BlockSpec((R, D), lambda i: (i, 0)),
          pl.BlockSpec((D, D), lambda i: (0, 0)),
          pl.BlockSpec((NC, R, 1), lambda i: (0, i, 0)),
      ],
      out_specs=[
          pl.BlockSpec((R, DH), lambda i: (i, 0)),
          pl.BlockSpec((R, DH), lambda i: (i, 0)),
          pl.BlockSpec((R, 1), lambda i: (i, 0)),
      ],
      out_shape=[
          jax.ShapeDtypeStruct((N, DH), jnp.float32),
          jax.ShapeDtypeStruct((N, DH), jnp.float32),
          jax.ShapeDtypeStruct((N, 1), jnp.float32),
      ],
  )(x, W1, degp)


def _tc_b_body(pa_ref, pb_ref, ga_ref, gb_ref, dinv_ref, b1_ref, w2_ref,
               h1_ref, ga2_ref, gb2_ref):
  pa = pa_ref[...]
  pb = pb_ref[...]
  dinv = dinv_ref[...]
  sa = pa[0] + pa[1] + ga_ref[...]
  sb = pb[0] + pb[1] + gb_ref[...]
  out1 = jnp.concatenate([sa, sb], axis=1) * dinv + b1_ref[...]
  h1 = jnp.maximum(out1, 0.0)
  h1_ref[...] = h1
  h2 = jnp.dot(h1, w2_ref[...], preferred_element_type=jnp.float32)
  g2 = h2 * dinv
  ga2_ref[...] = g2[:, :DH]
  gb2_ref[...] = g2[:, DH:]


def _tc_b(pa, pb, ga, gb, dinv, b1, W2):
  return pl.pallas_call(
      _tc_b_body,
      grid=(N // R,),
      in_specs=[
          pl.BlockSpec((NC, R, DH), lambda i: (0, i, 0)),
          pl.BlockSpec((NC, R, DH), lambda i: (0, i, 0)),
          pl.BlockSpec((R, DH), lambda i: (i, 0)),
          pl.BlockSpec((R, DH), lambda i: (i, 0)),
          pl.BlockSpec((R, 1), lambda i: (i, 0)),
          pl.BlockSpec((1, D), lambda i: (0, 0)),
          pl.BlockSpec((D, D), lambda i: (0, 0)),
      ],
      out_specs=[
          pl.BlockSpec((R, D), lambda i: (i, 0)),
          pl.BlockSpec((R, DH), lambda i: (i, 0)),
          pl.BlockSpec((R, DH), lambda i: (i, 0)),
      ],
      out_shape=[
          jax.ShapeDtypeStruct((N, D), jnp.float32),
          jax.ShapeDtypeStruct((N, DH), jnp.float32),
          jax.ShapeDtypeStruct((N, DH), jnp.float32),
      ],
  )(pa, pb, ga, gb, dinv, b1, W2)


def _tc_c_body(pa_ref, pb_ref, ga_ref, gb_ref, dinv_ref, b2_ref, out_ref):
  pa = pa_ref[...]
  pb = pb_ref[...]
  dinv = dinv_ref[...]
  sa = pa[0] + pa[1] + ga_ref[...]
  sb = pb[0] + pb[1] + gb_ref[...]
  out_ref[...] = jnp.concatenate([sa, sb], axis=1) * dinv + b2_ref[...]


def _tc_c(pa, pb, ga, gb, dinv, b2):
  return pl.pallas_call(
      _tc_c_body,
      grid=(N // R,),
      in_specs=[
          pl.BlockSpec((NC, R, DH), lambda i: (0, i, 0)),
          pl.BlockSpec((NC, R, DH), lambda i: (0, i, 0)),
          pl.BlockSpec((R, DH), lambda i: (i, 0)),
          pl.BlockSpec((R, DH), lambda i: (i, 0)),
          pl.BlockSpec((R, 1), lambda i: (i, 0)),
          pl.BlockSpec((1, D), lambda i: (0, 0)),
      ],
      out_specs=pl.BlockSpec((R, D), lambda i: (i, 0)),
      out_shape=jax.ShapeDtypeStruct((N, D), jnp.float32),
  )(pa, pb, ga, gb, dinv, b2)


@jax.jit
def kernel(x, adj_t, W1, b1, W2, b2):
  src = adj_t[0].astype(jnp.int32).reshape(NW, NCH, K)
  dst = adj_t[1].astype(jnp.int32).reshape(NW, NCH, K)
  zeros_hd = jnp.zeros((NP, DH), jnp.float32)
  zeros_n = jnp.zeros((N,), jnp.float32)
  ones_k = jnp.ones((K,), jnp.float32)

  degp = _deg_call(dst, ones_k, zeros_n)
  ga1, gb1, dinv = _tc_a(x, W1, degp.reshape(NC, N, 1))
  pa1, pb1 = _prop_call(ga1, gb1, src, dst, zeros_hd)
  h1, ga2, gb2 = _tc_b(pa1, pb1, ga1, gb1, dinv, b1.reshape(1, D), W2)
  pa2, pb2 = _prop_call(ga2, gb2, src, dst, zeros_hd)
  logit = _tc_c(pa2, pb2, ga2, gb2, dinv, b2.reshape(1, D))
  return (logit, h1)


# speedup vs baseline: 1.8046x; 1.0033x over previous
import functools

import jax
import jax.numpy as jnp
from jax import lax
from jax.experimental import pallas as pl
from jax.experimental.pallas import tpu as pltpu
from jax.experimental.pallas import tpu_sc as plsc

N = 10000
E = 320000
D = 128
DH = D // 2
NC = 2
NS = 16
NW = NC * NS
EPT = E // NW
K = 80
NCH = EPT // K
NP = 10112
RPS = NP // NS
NBUF = 13
ROUNDS = -(-NCH // NBUF)

_mesh = plsc.VectorSubcoreMesh(core_axis_name="c", subcore_axis_name="s")


def _deg_body(dst_hbm, ones_hbm, zeros_hbm, out_hbm, dst_v, ones_v, deg_sp):
  c = lax.axis_index("c")
  s = lax.axis_index("s")
  wid = c * NS + s
  @pl.when(s == 0)
  def _():
    pltpu.sync_copy(zeros_hbm, deg_sp)
  pltpu.sync_copy(ones_hbm, ones_v)
  pltpu.sync_copy(dst_hbm.at[wid], dst_v)
  plsc.subcore_barrier()

  def chunk(j, carry):
    pltpu.sync_copy(ones_v, deg_sp.at[dst_v.at[j]], add=True)
    return carry
  lax.fori_loop(0, NCH, chunk, 0)

  plsc.subcore_barrier()
  @pl.when(s == 0)
  def _():
    pltpu.sync_copy(deg_sp, out_hbm.at[c])


_deg_call = functools.partial(
    pl.kernel,
    out_type=jax.ShapeDtypeStruct((NC, N), jnp.float32),
    mesh=_mesh,
    scratch_types=[
        pltpu.VMEM((NCH, K), jnp.int32),
        pltpu.VMEM((K,), jnp.float32),
        pltpu.VMEM_SHARED((N,), jnp.float32),
    ],
)(_deg_body)


def _prop_body(ga_hbm, gb_hbm, src_hbm, dst_hbm, zeros_hbm,
               outa_hbm, outb_hbm, src_v, dst_v, *rest):
  rows = rest[:NBUF]
  acc_sp = rest[NBUF]
  gsems = rest[NBUF + 1:2 * NBUF + 1]
  ssems = rest[2 * NBUF + 1:]
  c = lax.axis_index("c")
  s = lax.axis_index("s")
  wid = c * NS + s
  pltpu.sync_copy(src_hbm.at[wid], src_v)
  pltpu.sync_copy(dst_hbm.at[wid], dst_v)

  for g_hbm, out_hbm in ((ga_hbm, outa_hbm), (gb_hbm, outb_hbm)):
    pltpu.sync_copy(zeros_hbm.at[pl.ds(s * RPS, RPS)],
                    acc_sp.at[pl.ds(s * RPS, RPS)])
    plsc.subcore_barrier()

    def gather_start(chunk, b):
      pltpu.async_copy(g_hbm.at[src_v.at[chunk]], rows[b], gsems[b])

    def gather_wait(b):
      pltpu.make_async_copy(g_hbm.at[src_v.at[0]], rows[b], gsems[b]).wait()

    def scatter_start(chunk, b):
      pltpu.async_copy(rows[b], acc_sp.at[dst_v.at[chunk]], ssems[b],
                       add=True)

    def scatter_wait(b):
      pltpu.make_async_copy(rows[b], acc_sp.at[dst_v.at[0]],
                            ssems[b]).wait()

    def round_body(rd, carry):
      base = rd * NBUF
      for b in range(NBUF):
        @pl.when(base + b < NCH)
        def _(b=b):
          gather_start(base + b, b)
      for b in range(NBUF):
        @pl.when(base + b < NCH)
        def _(b=b):
          gather_wait(b)
          scatter_start(base + b, b)
      for b in range(NBUF):
        @pl.when(base + b < NCH)
        def _(b=b):
          scatter_wait(b)
      return carry
    lax.fori_loop(0, ROUNDS, round_body, 0)

    plsc.subcore_barrier()
    pltpu.sync_copy(acc_sp.at[pl.ds(s * RPS, RPS)],
                    out_hbm.at[c, pl.ds(s * RPS, RPS)])
    plsc.subcore_barrier()


_prop_call = functools.partial(
    pl.kernel,
    out_type=[
        jax.ShapeDtypeStruct((NC, NP, DH), jnp.float32),
        jax.ShapeDtypeStruct((NC, NP, DH), jnp.float32),
    ],
    mesh=_mesh,
    compiler_params=pltpu.CompilerParams(use_tc_tiling_on_sc=False),
    scratch_types=(
        [pltpu.VMEM((NCH, K), jnp.int32),
         pltpu.VMEM((NCH, K), jnp.int32)]
        + [pltpu.VMEM((K, DH), jnp.float32) for _ in range(NBUF)]
        + [pltpu.VMEM_SHARED((NP, DH), jnp.float32)]
        + [pltpu.SemaphoreType.DMA for _ in range(2 * NBUF)]
    ),
)(_prop_body)


R = 5000


def _tc_a_body(x_ref, w_ref, degp_ref, ga_ref, gb_ref, dinv_ref):
  d = degp_ref[...]
  deg = 1.0 + d[0] + d[1]
  dinv = lax.rsqrt(deg)
  h = jnp.dot(x_ref[...], w_ref[...], preferred_element_type=jnp.float32)
  g = h * dinv
  ga_ref[...] = g[:, :DH]
  gb_ref[...] = g[:, DH:]
  dinv_ref[...] = dinv


def _tc_a(x, W1, degp):
  return pl.pallas_call(
      _tc_a_body,
      grid=(N // R,),
      in_specs=[
          pl.BlockSpec((R, D), lambda i: (i, 0)),
          pl.BlockSpec((D, D), lambda i: (0, 0)),
          pl.BlockSpec((NC, R, 1), lambda i: (0, i, 0)),
      ],
      out_specs=[
          pl.BlockSpec((R, DH), lambda i: (i, 0)),
          pl.BlockSpec((R, DH), lambda i: (i, 0)),
          pl.BlockSpec((R, 1), lambda i: (i, 0)),
      ],
      out_shape=[
          jax.ShapeDtypeStruct((N, DH), jnp.float32),
          jax.ShapeDtypeStruct((N, DH), jnp.float32),
          jax.ShapeDtypeStruct((N, 1), jnp.float32),
      ],
  )(x, W1, degp)


def _tc_b_body(pa_ref, pb_ref, ga_ref, gb_ref, dinv_ref, b1_ref, w2_ref,
               h1_ref, ga2_ref, gb2_ref):
  pa = pa_ref[...]
  pb = pb_ref[...]
  dinv = dinv_ref[...]
  sa = pa[0] + pa[1] + ga_ref[...]
  sb = pb[0] + pb[1] + gb_ref[...]
  out1 = jnp.concatenate([sa, sb], axis=1) * dinv + b1_ref[...]
  h1 = jnp.maximum(out1, 0.0)
  h1_ref[...] = h1
  h2 = jnp.dot(h1, w2_ref[...], preferred_element_type=jnp.float32)
  g2 = h2 * dinv
  ga2_ref[...] = g2[:, :DH]
  gb2_ref[...] = g2[:, DH:]


def _tc_b(pa, pb, ga, gb, dinv, b1, W2):
  return pl.pallas_call(
      _tc_b_body,
      grid=(N // R,),
      in_specs=[
          pl.BlockSpec((NC, R, DH), lambda i: (0, i, 0)),
          pl.BlockSpec((NC, R, DH), lambda i: (0, i, 0)),
          pl.BlockSpec((R, DH), lambda i: (i, 0)),
          pl.BlockSpec((R, DH), lambda i: (i, 0)),
          pl.BlockSpec((R, 1), lambda i: (i, 0)),
          pl.BlockSpec((1, D), lambda i: (0, 0)),
          pl.BlockSpec((D, D), lambda i: (0, 0)),
      ],
      out_specs=[
          pl.BlockSpec((R, D), lambda i: (i, 0)),
          pl.BlockSpec((R, DH), lambda i: (i, 0)),
          pl.BlockSpec((R, DH), lambda i: (i, 0)),
      ],
      out_shape=[
          jax.ShapeDtypeStruct((N, D), jnp.float32),
          jax.ShapeDtypeStruct((N, DH), jnp.float32),
          jax.ShapeDtypeStruct((N, DH), jnp.float32),
      ],
  )(pa, pb, ga, gb, dinv, b1, W2)


def _tc_c_body(pa_ref, pb_ref, ga_ref, gb_ref, dinv_ref, b2_ref, out_ref):
  pa = pa_ref[...]
  pb = pb_ref[...]
  dinv = dinv_ref[...]
  sa = pa[0] + pa[1] + ga_ref[...]
  sb = pb[0] + pb[1] + gb_ref[...]
  out_ref[...] = jnp.concatenate([sa, sb], axis=1) * dinv + b2_ref[...]


def _tc_c(pa, pb, ga, gb, dinv, b2):
  return pl.pallas_call(
      _tc_c_body,
      grid=(N // R,),
      in_specs=[
          pl.BlockSpec((NC, R, DH), lambda i: (0, i, 0)),
          pl.BlockSpec((NC, R, DH), lambda i: (0, i, 0)),
          pl.BlockSpec((R, DH), lambda i: (i, 0)),
          pl.BlockSpec((R, DH), lambda i: (i, 0)),
          pl.BlockSpec((R, 1), lambda i: (i, 0)),
          pl.BlockSpec((1, D), lambda i: (0, 0)),
      ],
      out_specs=pl.BlockSpec((R, D), lambda i: (i, 0)),
      out_shape=jax.ShapeDtypeStruct((N, D), jnp.float32),
  )(pa, pb, ga, gb, dinv, b2)


@jax.jit
def kernel(x, adj_t, W1, b1, W2, b2):
  src = adj_t[0].astype(jnp.int32).reshape(NW, NCH, K)
  dst = adj_t[1].astype(jnp.int32).reshape(NW, NCH, K)
  zeros_hd = jnp.zeros((NP, DH), jnp.float32)
  zeros_n = jnp.zeros((N,), jnp.float32)
  ones_k = jnp.ones((K,), jnp.float32)

  degp = _deg_call(dst, ones_k, zeros_n)
  ga1, gb1, dinv = _tc_a(x, W1, degp.reshape(NC, N, 1))
  pa1, pb1 = _prop_call(ga1, gb1, src, dst, zeros_hd)
  h1, ga2, gb2 = _tc_b(pa1, pb1, ga1, gb1, dinv, b1.reshape(1, D), W2)
  pa2, pb2 = _prop_call(ga2, gb2, src, dst, zeros_hd)
  logit = _tc_c(pa2, pb2, ga2, gb2, dinv, b2.reshape(1, D))
  return (logit, h1)
